# baseline (speedup 1.0000x reference)
import jax
import jax.numpy as jnp
from jax import lax
from jax.experimental import pallas as pl
from jax.experimental.pallas import tpu as pltpu
from jax.experimental.pallas import tpu_sc as plsc

_VOCAB = 50304
_MAX_POS = 8192
_EMBED = 768
_BATCH = 4
_SEQ = 8192

_NC = 2
_NS = 16
_NW = _NC * _NS
_POS_PER_W = _SEQ // _NW
_C = 32
_NCHUNK = _POS_PER_W // _C
_NVREG = _EMBED // 16


def _body(ids_hbm, w_word, w_pos, out_hbm,
          idx_v, rbuf0, rbuf1, rbuf2, rbuf3, posbuf,
          sem_g0, sem_g1, sem_g2, sem_g3,
          sem_s0, sem_s1, sem_s2, sem_s3, sem_p):
    cid = lax.axis_index("c")
    sid = lax.axis_index("s")
    wid = sid * _NC + cid
    pos0 = wid * _POS_PER_W

    rbufs = (rbuf0, rbuf1, rbuf2, rbuf3)
    gsems = (sem_g0, sem_g1, sem_g2, sem_g3)
    ssems = (sem_s0, sem_s1, sem_s2, sem_s3)

    def gather(b_idx, off, rb, sem):
        pltpu.async_copy(w_word.at[idx_v.at[b_idx, pl.ds(off, _C)]], rb, sem)

    def wait_tile(rb, sem):
        pltpu.make_async_copy(w_pos.at[pl.ds(0, _C)], rb, sem).wait()

    def load_pos(g, sem):
        pltpu.async_copy(w_pos.at[pl.ds(pos0 + g * _C, _C)], posbuf, sem)

    pltpu.sync_copy(ids_hbm.at[:, pl.ds(pos0, _POS_PER_W)], idx_v)

    load_pos(0, sem_p)
    gather(0, 0, rbuf0, sem_g0)
    gather(1, 0, rbuf1, sem_g1)

    def tile_ops(g, gnxt, b, first):
        n2 = (b + 2) % _BATCH

        if not (first and b < 2):
            wait_tile(rbufs[n2], ssems[n2])

        if b < 2:
            gather(b + 2, g * _C, rbufs[n2], gsems[n2])
        elif first:
            gather(b - 2, gnxt * _C, rbufs[n2], gsems[n2])
        else:
            @pl.when(g < _NCHUNK - 1)
            def _gather_next_chunk():
                gather(b - 2, gnxt * _C, rbufs[n2], gsems[n2])

        if b == 0:
            wait_tile(posbuf, sem_p)

        wait_tile(rbufs[b], gsems[b])

        def add_row(r, carry):
            for j in range(_NVREG):
                s = pl.ds(j * 16, 16)
                rbufs[b][r, s] = rbufs[b][r, s] + posbuf[r, s]
            return carry

        lax.fori_loop(0, _C, add_row, 0, unroll=False)

        if b == _BATCH - 1:
            if first:
                load_pos(gnxt, sem_p)
            else:
                @pl.when(g < _NCHUNK - 1)
                def _prefetch_pos():
                    load_pos(gnxt, sem_p)

        pltpu.async_copy(
            rbufs[b],
            out_hbm.at[pl.ds(b * _SEQ + pos0 + g * _C, _C)],
            ssems[b])

    for b in range(_BATCH):
        tile_ops(0, 1, b, first=True)

    def chunk_body(g, _):
        for b in range(_BATCH):
            tile_ops(g, lax.rem(g + 1, _NCHUNK), b, first=False)
        return _

    lax.fori_loop(1, _NCHUNK, chunk_body, 0, unroll=False)

    wait_tile(rbuf2, sem_s2)
    wait_tile(rbuf3, sem_s3)


@jax.jit
def _embed(input_ids, w_word, w_pos):
    mesh = plsc.VectorSubcoreMesh(core_axis_name="c", subcore_axis_name="s")
    k = pl.kernel(
        _body,
        out_type=jax.ShapeDtypeStruct((_BATCH * _SEQ, _EMBED), jnp.float32),
        mesh=mesh,
        scratch_types=[
            pltpu.VMEM((_BATCH, _POS_PER_W), jnp.int32),
            pltpu.VMEM((_C, _EMBED), jnp.float32),
            pltpu.VMEM((_C, _EMBED), jnp.float32),
            pltpu.VMEM((_C, _EMBED), jnp.float32),
            pltpu.VMEM((_C, _EMBED), jnp.float32),
            pltpu.VMEM((_C, _EMBED), jnp.float32),
            pltpu.SemaphoreType.DMA,
            pltpu.SemaphoreType.DMA,
            pltpu.SemaphoreType.DMA,
            pltpu.SemaphoreType.DMA,
            pltpu.SemaphoreType.DMA,
            pltpu.SemaphoreType.DMA,
            pltpu.SemaphoreType.DMA,
            pltpu.SemaphoreType.DMA,
            pltpu.SemaphoreType.DMA,
        ],
    )
    return k(input_ids, w_word, w_pos)


def kernel(input_ids, W_word, W_pos):
    ids = input_ids.astype(jnp.int32)
    out = _embed(ids, W_word, W_pos)
    return out.reshape(_BATCH, _SEQ, _EMBED)
